# baseline (speedup 1.0000x reference)
import functools

import jax
import jax.numpy as jnp
from jax.experimental import pallas as pl
from jax.experimental.pallas import tpu as pltpu

_GAM1 = 0.01
_GAM2 = 0.01


def _pge_tc_kernel(p_ref, p2_ref, q_ref, q2_ref, lab_ref, out_ref, *,
                   n_classes, n_pivots, n_total):
    i = pl.program_id(0)
    q = q_ref[...]
    bm = q.shape[0]

    t = jax.lax.dot_general(
        p_ref[...], q, (((1,), (1,)), ((), ())),
        preferred_element_type=jnp.float32)
    t = t + p2_ref[...]
    t3 = t.reshape(n_classes, n_pivots, bm)
    mn = jnp.min(t3, axis=1)
    mx = jnp.max(t3, axis=1)

    q2 = q2_ref[0:1, :]
    mind = jnp.sqrt(jnp.maximum(mn + q2, 1e-12))
    maxd = jnp.sqrt(jnp.maximum(mx + q2, 1e-12))

    cls = jax.lax.broadcasted_iota(jnp.int32, (n_classes, bm), 0)
    own = lab_ref[0:1, :] == cls

    s_all_min = jnp.sum(mind)
    s_own_min = jnp.sum(jnp.where(own, mind, 0.0))
    s_own_max = jnp.sum(jnp.where(own, maxd, 0.0))

    part = (_GAM1 / n_total) * s_own_max \
        - (_GAM2 / (n_total * (n_classes - 1))) * (s_all_min - s_own_min)

    @pl.when(i == 0)
    def _init():
        out_ref[0, 0] = jnp.float32(0.0)

    out_ref[0, 0] += part


def kernel(queries, pivots, labels):
    B, d = queries.shape
    C, Np, _ = pivots.shape
    bm = 256

    p = pivots.reshape(C * Np, d)
    p_neg2 = (-2.0 * p).astype(jnp.bfloat16)
    p2 = jnp.sum(p * p, axis=-1, keepdims=True)
    q_bf = queries.astype(jnp.bfloat16)
    q2 = jnp.broadcast_to(
        jnp.sum(queries * queries, axis=-1)[None, :], (8, B))
    lab = jnp.broadcast_to(
        labels.astype(jnp.int32)[None, :], (8, B))

    grid = (B // bm,)
    out = pl.pallas_call(
        functools.partial(_pge_tc_kernel, n_classes=C, n_pivots=Np,
                          n_total=B),
        grid=grid,
        in_specs=[
            pl.BlockSpec((C * Np, d), lambda i: (0, 0)),
            pl.BlockSpec((C * Np, 1), lambda i: (0, 0)),
            pl.BlockSpec((bm, d), lambda i: (i, 0)),
            pl.BlockSpec((8, bm), lambda i: (0, i)),
            pl.BlockSpec((8, bm), lambda i: (0, i)),
        ],
        out_specs=pl.BlockSpec(memory_space=pltpu.SMEM),
        out_shape=jax.ShapeDtypeStruct((1, 1), jnp.float32),
        compiler_params=pltpu.CompilerParams(
            dimension_semantics=("arbitrary",)),
    )(p_neg2, p2, q_bf, q2, lab)
    return out[0, 0]


# speedup vs baseline: 2.0703x; 2.0703x over previous
import functools

import jax
import jax.numpy as jnp
from jax.experimental import pallas as pl
from jax.experimental.pallas import tpu as pltpu

_GAM1 = 0.01
_GAM2 = 0.01


def _pge_tc_kernel(q_ref, p_ref, p2_ref, lab_ref, out_ref, *,
                   n_classes, n_total, np_):
    i = pl.program_id(0)
    q = q_ref[...]
    bm = q.shape[0]
    q2 = jnp.sum(q * q, axis=1, keepdims=True)
    qm2 = (-2.0 * q).astype(jnp.bfloat16)

    big = jnp.float32(3.0e38)
    mn = jnp.full((bm, n_classes), big, jnp.bfloat16)
    mx = jnp.full((bm, n_classes), -big, jnp.bfloat16)
    for k in range(np_):
        qp = jax.lax.dot_general(
            qm2, p_ref[k], (((1,), (1,)), ((), ())),
            preferred_element_type=jnp.float32)
        t = qp.astype(jnp.bfloat16) + p2_ref[k]
        mn = jnp.minimum(mn, t)
        mx = jnp.maximum(mx, t)

    mind = jnp.sqrt(jnp.maximum(mn.astype(jnp.float32) + q2, 1e-12))
    maxd = jnp.sqrt(jnp.maximum(mx.astype(jnp.float32) + q2, 1e-12))

    cls = jax.lax.broadcasted_iota(jnp.int32, (bm, n_classes), 1)
    own = lab_ref[...] == cls

    s_all_min = jnp.sum(mind)
    s_own_min = jnp.sum(jnp.where(own, mind, 0.0))
    s_own_max = jnp.sum(jnp.where(own, maxd, 0.0))

    part = (_GAM1 / n_total) * s_own_max \
        - (_GAM2 / (n_total * (n_classes - 1))) * (s_all_min - s_own_min)

    @pl.when(i == 0)
    def _init():
        out_ref[0, 0] = jnp.float32(0.0)

    out_ref[0, 0] += part


def kernel(queries, pivots, labels):
    B, d = queries.shape
    C, Np, _ = pivots.shape
    bm = 256

    p_t = jnp.transpose(pivots.astype(jnp.bfloat16), (1, 0, 2))
    p2 = jnp.sum(pivots * pivots, axis=-1)
    p2 = jnp.transpose(p2, (1, 0))[:, None, :].astype(jnp.bfloat16)
    lab = labels.astype(jnp.int32).reshape(B, 1)

    grid = (B // bm,)
    out = pl.pallas_call(
        functools.partial(_pge_tc_kernel, n_classes=C, n_total=B, np_=Np),
        grid=grid,
        in_specs=[
            pl.BlockSpec((bm, d), lambda i: (i, 0)),
            pl.BlockSpec((Np, C, d), lambda i: (0, 0, 0)),
            pl.BlockSpec((Np, 1, C), lambda i: (0, 0, 0)),
            pl.BlockSpec((bm, 1), lambda i: (i, 0)),
        ],
        out_specs=pl.BlockSpec(memory_space=pltpu.SMEM),
        out_shape=jax.ShapeDtypeStruct((1, 1), jnp.float32),
        compiler_params=pltpu.CompilerParams(
            dimension_semantics=("arbitrary",)),
    )(queries, p_t, p2, lab)
    return out[0, 0]


# speedup vs baseline: 2.0740x; 1.0018x over previous
import functools

import jax
import jax.numpy as jnp
from jax.experimental import pallas as pl
from jax.experimental.pallas import tpu as pltpu

_GAM1 = 0.01
_GAM2 = 0.01


def _pge_tc_kernel(q_ref, p_ref, lab_ref, out_ref, p2_scr, *,
                   n_classes, n_total, np_):
    i = pl.program_id(0)

    @pl.when(i == 0)
    def _precompute_pivot_norms():
        ones8 = jnp.ones((8, q_ref.shape[1]), jnp.bfloat16)
        for k in range(np_):
            pk = p_ref[k]
            p2k = jax.lax.dot_general(
                ones8, pk * pk, (((1,), (1,)), ((), ())),
                preferred_element_type=jnp.float32)
            p2_scr[k] = p2k.astype(jnp.bfloat16)
    q = q_ref[...]
    bm = q.shape[0]
    q2 = jnp.sum(q * q, axis=1, keepdims=True)
    qm2 = (-2.0 * q).astype(jnp.bfloat16)

    big = jnp.float32(3.0e38)
    mn = jnp.full((bm, n_classes), big, jnp.bfloat16)
    mx = jnp.full((bm, n_classes), -big, jnp.bfloat16)
    for k in range(np_):
        qp = jax.lax.dot_general(
            qm2, p_ref[k], (((1,), (1,)), ((), ())),
            preferred_element_type=jnp.float32)
        t = qp.astype(jnp.bfloat16) + p2_scr[k, 0:1, :]
        mn = jnp.minimum(mn, t)
        mx = jnp.maximum(mx, t)

    mind = jnp.sqrt(jnp.maximum(mn.astype(jnp.float32) + q2, 1e-12))
    maxd = jnp.sqrt(jnp.maximum(mx.astype(jnp.float32) + q2, 1e-12))

    cls = jax.lax.broadcasted_iota(jnp.int32, (bm, n_classes), 1)
    own = lab_ref[...] == cls

    s_all_min = jnp.sum(mind)
    s_own_min = jnp.sum(jnp.where(own, mind, 0.0))
    s_own_max = jnp.sum(jnp.where(own, maxd, 0.0))

    part = (_GAM1 / n_total) * s_own_max \
        - (_GAM2 / (n_total * (n_classes - 1))) * (s_all_min - s_own_min)

    @pl.when(i == 0)
    def _init():
        out_ref[0, 0] = jnp.float32(0.0)

    out_ref[0, 0] += part


def kernel(queries, pivots, labels):
    B, d = queries.shape
    C, Np, _ = pivots.shape
    bm = 256

    p_t = jnp.transpose(pivots.astype(jnp.bfloat16), (1, 0, 2))
    lab = labels.astype(jnp.int32).reshape(B, 1)

    grid = (B // bm,)
    out = pl.pallas_call(
        functools.partial(_pge_tc_kernel, n_classes=C, n_total=B, np_=Np),
        grid=grid,
        in_specs=[
            pl.BlockSpec((bm, d), lambda i: (i, 0)),
            pl.BlockSpec((Np, C, d), lambda i: (0, 0, 0)),
            pl.BlockSpec((bm, 1), lambda i: (i, 0)),
        ],
        scratch_shapes=[pltpu.VMEM((Np, 8, C), jnp.bfloat16)],
        out_specs=pl.BlockSpec(memory_space=pltpu.SMEM),
        out_shape=jax.ShapeDtypeStruct((1, 1), jnp.float32),
        compiler_params=pltpu.CompilerParams(
            dimension_semantics=("arbitrary",)),
    )(queries, p_t, lab)
    return out[0, 0]
